# Initial kernel scaffold; baseline (speedup 1.0000x reference)
import jax
import jax.numpy as jnp
from jax.experimental import pallas as pl


def kernel(emb, x, freq_table, phase_table):
    raise NotImplementedError("write your pallas kernel here")


# speedup vs baseline: 3.2457x; 3.2457x over previous
import functools
import math

import jax
import jax.numpy as jnp
import numpy as np
from jax import lax
from jax.experimental import pallas as pl
from jax.experimental.pallas import tpu as pltpu
from jax.experimental.pallas import tpu_sc as plsc

_TWO_PI = 2.0 * math.pi
_INV_2PI = 1.0 / _TWO_PI
_MAGIC = 1.5 * 2.0 ** 23
_TPI_HI = float(np.float32(_TWO_PI))
_TPI_LO = _TWO_PI - _TPI_HI
_COS_COEF = (9.99999211e-01, -4.99994213e-01, 4.16597778e-02,
             -1.38587899e-03, 2.42029414e-05, -2.19729638e-07)
_SIN_COEF = (9.99999600e-01, -1.66665526e-01, 8.33240299e-03,
             -1.98086333e-04, 2.69971464e-06, -2.03622449e-08)


def _sc_gather_body(comb_hbm, idx_hbm, out_hbm, idx_v, comb_v, sem, *,
                    per_w, ch, n_chunks, nc):
    wid = lax.axis_index("s") * nc + lax.axis_index("c")
    base = wid * per_w
    for j in range(n_chunks):
        off = base + j * ch
        pltpu.sync_copy(idx_hbm.at[pl.ds(off, ch)], idx_v)
        pltpu.async_copy(comb_hbm.at[idx_v], comb_v, sem).wait()
        pltpu.sync_copy(comb_v, out_hbm.at[pl.ds(off, ch)])


def _sc_gather(comb, idx_flat):
    n_total = idx_flat.shape[0]
    d2 = comb.shape[1]
    info = plsc.get_sparse_core_info()
    nc, ns = info.num_cores, info.num_subcores
    nw = nc * ns
    per_w = n_total // nw
    ch = 400
    n_chunks = per_w // ch
    assert per_w % ch == 0 and n_total % nw == 0

    mesh = plsc.VectorSubcoreMesh(core_axis_name="c", subcore_axis_name="s")
    kern = pl.kernel(
        functools.partial(_sc_gather_body, per_w=per_w, ch=ch,
                          n_chunks=n_chunks, nc=nc),
        mesh=mesh,
        out_type=jax.ShapeDtypeStruct((n_total, d2), jnp.float32),
        scratch_types=[
            pltpu.VMEM((ch,), jnp.int32),
            pltpu.VMEM((ch, d2), jnp.float32),
            pltpu.SemaphoreType.DMA,
        ],
    )
    return kern(comb, idx_flat)


def _tc_body(c_ref, e_ref, real_ref, imag_ref, *, blk, b_per_s, d):
    i = pl.program_id(0)
    pos = ((lax.broadcasted_iota(jnp.int32, (blk, d), 0) + i * blk)
           // b_per_s + 1).astype(jnp.float32)
    c = c_ref[...]
    freq = c[:, :d]
    bias = c[:, d:]
    phase = pos * freq + bias
    n = jnp.round(phase * _INV_2PI)
    r = (phase - n * _TPI_HI) - n * _TPI_LO
    r2 = r * r
    rr2 = jnp.concatenate([r2, r2], axis=1)
    left = lax.broadcasted_iota(jnp.int32, (1, 2 * d), 1) < d
    poly = jnp.where(left, _COS_COEF[5], _SIN_COEF[5]).astype(jnp.float32)
    for k in (4, 3, 2, 1, 0):
        ck = jnp.where(left, _COS_COEF[k], _SIN_COEF[k]).astype(jnp.float32)
        poly = poly * rr2 + ck
    e = e_ref[...]
    m = jnp.concatenate([e, e * r], axis=1)
    out = m * poly
    real_ref[...] = out[:, :d]
    imag_ref[...] = out[:, d:]


def _tc_pointwise(comb_rows, e_flat, b_per_s):
    n, d2 = comb_rows.shape
    d = d2 // 2
    blk = 2048
    grid = (n // blk,)
    spec_c = pl.BlockSpec((blk, d2), lambda i: (i, 0))
    spec_e = pl.BlockSpec((blk, d), lambda i: (i, 0))
    return pl.pallas_call(
        functools.partial(_tc_body, blk=blk, b_per_s=b_per_s, d=d),
        grid=grid,
        in_specs=[spec_c, spec_e],
        out_specs=[spec_e, spec_e],
        out_shape=[
            jax.ShapeDtypeStruct((n, d), jnp.float32),
            jax.ShapeDtypeStruct((n, d), jnp.float32),
        ],
    )(comb_rows, e_flat)


def kernel(emb, x, freq_table, phase_table):
    s, b, d = emb.shape
    comb = jnp.concatenate([freq_table, phase_table], axis=1)
    idx_flat = x.reshape(-1)
    comb_rows = _sc_gather(comb, idx_flat)
    real, imag = _tc_pointwise(comb_rows, emb.reshape(s * b, d), b)
    return real.reshape(s, b, d), imag.reshape(s, b, d)
